# baseline (speedup 1.0000x reference)
import functools

import jax
import jax.numpy as jnp
from jax import lax
from jax.experimental import pallas as pl
from jax.experimental.pallas import tpu as pltpu
from jax.experimental.pallas import tpu_sc as plsc

B, S, D = 16, 4096, 1024
NC, NS, LANES = 2, 16, 16
NW = NC * NS
ROWS_PER_W = (B * S) // NW
CH = 32
NCHUNK = ROWS_PER_W // CH


def _make_sc_reverse():
    mesh = plsc.VectorSubcoreMesh(core_axis_name="c", subcore_axis_name="s")

    @functools.partial(
        pl.kernel,
        mesh=mesh,
        out_type=jax.ShapeDtypeStruct((B * S, D), jnp.float32),
        scratch_types=[
            pltpu.VMEM((B * LANES,), jnp.int32),
            pltpu.VMEM((ROWS_PER_W,), jnp.int32),
            pltpu.VMEM((CH, D), jnp.float32),
            pltpu.VMEM((CH, D), jnp.float32),
            pltpu.SemaphoreType.DMA,
            pltpu.SemaphoreType.DMA,
            pltpu.SemaphoreType.DMA,
            pltpu.SemaphoreType.DMA,
            pltpu.SemaphoreType.DMA,
        ],
    )
    def k(x_hbm, len_hbm, out_hbm, len_v, idx_v, buf0, buf1,
          sg0, sg1, ss0, ss1, sd):
        wid = lax.axis_index("s") * NC + lax.axis_index("c")
        b = wid // 2
        s_base = (wid % 2) * ROWS_PER_W
        row0 = b * S

        pltpu.sync_copy(len_hbm, len_v)
        lvec = len_v[pl.ds(b * LANES, LANES)][0]

        def build(kk, _):
            p = s_base + kk * LANES + lax.iota(jnp.int32, LANES)
            src = jnp.where(p < lvec, lvec - 1 - p, p) + row0
            idx_v[pl.ds(kk * LANES, LANES)] = src
            return 0

        lax.fori_loop(0, ROWS_PER_W // LANES, build, 0)

        out0 = row0 + s_base
        bufs = (buf0, buf1)
        g_sems = (sg0, sg1)
        s_sems = (ss0, ss1)

        def gather(c, j):
            return pltpu.make_async_copy(
                x_hbm.at[idx_v.at[pl.ds(c * CH, CH)]], bufs[j], g_sems[j])

        def scatter(c, j):
            return pltpu.make_async_copy(
                bufs[j], out_hbm.at[pl.ds(out0 + c * CH, CH)], s_sems[j])

        def direct(c):
            return pltpu.make_async_copy(
                x_hbm.at[pl.ds(out0 + c * CH, CH)],
                out_hbm.at[pl.ds(out0 + c * CH, CH)], sd)

        def needs_rev(c):
            return s_base + c * CH < lvec

        def step(c, j):
            @pl.when(needs_rev(c))
            def _():
                gather(c, j).wait()
                scatter(c, j).start()

            @pl.when(jnp.logical_not(needs_rev(c)))
            def _():
                direct(c).start()

            @pl.when(jnp.logical_and(c >= 1, needs_rev(c - 1)))
            def _():
                scatter(c - 1, 1 - j).wait()

            @pl.when(jnp.logical_and(c + 1 < NCHUNK, needs_rev(c + 1)))
            def _():
                gather(c + 1, 1 - j).start()

        @pl.when(needs_rev(0))
        def _():
            gather(0, 0).start()

        def pair(i, _):
            step(2 * i, 0)
            step(2 * i + 1, 1)
            return 0

        lax.fori_loop(0, NCHUNK // 2, pair, 0)

        @pl.when(needs_rev(NCHUNK - 1))
        def _():
            scatter(NCHUNK - 1, (NCHUNK - 1) % 2).wait()

        def drain(c, _):
            @pl.when(jnp.logical_not(needs_rev(c)))
            def _():
                direct(c).wait()
            return 0

        lax.fori_loop(0, NCHUNK, drain, 0)

    return k


def kernel(x, seq_lengths):
    x2 = x.reshape(B * S, D)
    lens = jnp.repeat(seq_lengths.astype(jnp.int32), LANES)
    out = _make_sc_reverse()(x2, lens)
    return out.reshape(B, S, D)


# speedup vs baseline: 24.1798x; 24.1798x over previous
import functools

import jax
import jax.numpy as jnp
from jax import lax
from jax.experimental import pallas as pl
from jax.experimental.pallas import tpu as pltpu
from jax.experimental.pallas import tpu_sc as plsc

B, S, D = 16, 4096, 1024
NC, NS, LANES = 2, 16, 16
NW = NC * NS
ROWS_PER_W = (B * S) // NW
CH = 32
NCHUNK = ROWS_PER_W // CH


def _make_sc_reverse():
    mesh = plsc.VectorSubcoreMesh(core_axis_name="c", subcore_axis_name="s")

    @functools.partial(
        pl.kernel,
        mesh=mesh,
        out_type=jax.ShapeDtypeStruct((B * S, D), jnp.float32),
        scratch_types=[
            pltpu.VMEM((B * LANES,), jnp.int32),
            pltpu.VMEM((ROWS_PER_W,), jnp.int32),
            pltpu.VMEM((CH, D), jnp.float32),
            pltpu.VMEM((CH, D), jnp.float32),
            pltpu.SemaphoreType.DMA,
            pltpu.SemaphoreType.DMA,
            pltpu.SemaphoreType.DMA,
            pltpu.SemaphoreType.DMA,
        ],
    )
    def k(x_hbm, len_hbm, out_hbm, len_v, idx_v, buf0, buf1,
          sg0, sg1, ss0, ss1):
        wid = lax.axis_index("s") * NC + lax.axis_index("c")
        b = wid // 2
        s_base = (wid % 2) * ROWS_PER_W
        row0 = b * S

        pltpu.sync_copy(len_hbm, len_v)
        lvec = len_v[pl.ds(b * LANES, LANES)][0]

        def build(kk, _):
            p = s_base + kk * LANES + lax.iota(jnp.int32, LANES)
            src = jnp.where(p < lvec, lvec - 1 - p, p) + row0
            idx_v[pl.ds(kk * LANES, LANES)] = src
            return 0

        lax.fori_loop(0, ROWS_PER_W // LANES, build, 0)

        out0 = row0 + s_base
        bufs = (buf0, buf1)
        g_sems = (sg0, sg1)
        s_sems = (ss0, ss1)

        def gather(c, j):
            return pltpu.make_async_copy(
                x_hbm.at[idx_v.at[pl.ds(c * CH, CH)]], bufs[j], g_sems[j])

        def scatter(c, j):
            return pltpu.make_async_copy(
                bufs[j], out_hbm.at[pl.ds(out0 + c * CH, CH)], s_sems[j])

        def step(c, j):
            gather(c, j).wait()
            scatter(c, j).start()

            @pl.when(c >= 1)
            def _():
                scatter(c - 1, 1 - j).wait()

            @pl.when(c + 1 < NCHUNK)
            def _():
                gather(c + 1, 1 - j).start()

        gather(0, 0).start()

        def pair(i, _):
            step(2 * i, 0)
            step(2 * i + 1, 1)
            return 0

        lax.fori_loop(0, NCHUNK // 2, pair, 0)
        scatter(NCHUNK - 1, (NCHUNK - 1) % 2).wait()

    return k


def kernel(x, seq_lengths):
    x2 = x.reshape(B * S, D)
    lens = jnp.repeat(seq_lengths.astype(jnp.int32), LANES)
    out = _make_sc_reverse()(x2, lens)
    return out.reshape(B, S, D)


# speedup vs baseline: 24.2150x; 1.0015x over previous
import functools

import jax
import jax.numpy as jnp
from jax import lax
from jax.experimental import pallas as pl
from jax.experimental.pallas import tpu as pltpu
from jax.experimental.pallas import tpu_sc as plsc

B, S, D = 16, 4096, 1024
NC, NS, LANES = 2, 16, 16
NW = NC * NS
ROWS_PER_W = (B * S) // NW
CH = 32
NCHUNK = ROWS_PER_W // CH
NBUF = 3


def _make_sc_reverse():
    mesh = plsc.VectorSubcoreMesh(core_axis_name="c", subcore_axis_name="s")

    @functools.partial(
        pl.kernel,
        mesh=mesh,
        out_type=jax.ShapeDtypeStruct((B * S, D), jnp.float32),
        scratch_types=[
            pltpu.VMEM((B * LANES,), jnp.int32),
            pltpu.VMEM((ROWS_PER_W,), jnp.int32),
            pltpu.VMEM((NBUF, CH, D), jnp.float32),
            pltpu.SemaphoreType.DMA((NBUF,)),
            pltpu.SemaphoreType.DMA((NBUF,)),
        ],
    )
    def k(x_hbm, len_hbm, out_hbm, len_v, idx_v, buf, sg, ss):
        wid = lax.axis_index("s") * NC + lax.axis_index("c")
        b = wid // 2
        s_base = (wid % 2) * ROWS_PER_W
        row0 = b * S

        pltpu.sync_copy(len_hbm, len_v)
        lvec = len_v[pl.ds(b * LANES, LANES)][0]
        out0 = row0 + s_base

        def build(cc):
            for r in range(CH // LANES):
                p = s_base + cc * CH + r * LANES + lax.iota(jnp.int32, LANES)
                src = jnp.where(p < lvec, lvec - 1 - p, p) + row0
                idx_v[pl.ds(cc * CH + r * LANES, LANES)] = src

        def gather(c):
            j = c % NBUF
            return pltpu.make_async_copy(
                x_hbm.at[idx_v.at[pl.ds(c * CH, CH)]], buf.at[j], sg.at[j])

        def scatter(c):
            j = c % NBUF
            return pltpu.make_async_copy(
                buf.at[j], out_hbm.at[pl.ds(out0 + c * CH, CH)], ss.at[j])

        build(0)
        gather(0).start()
        build(1)
        gather(1).start()

        def step(c, _):
            gather(c).wait()
            scatter(c).start()

            @pl.when(c >= 1)
            def _():
                scatter(c - 1).wait()

            @pl.when(c + 2 < NCHUNK)
            def _():
                build(c + 2)
                gather(c + 2).start()

            return 0

        lax.fori_loop(0, NCHUNK, step, 0)
        scatter(NCHUNK - 1).wait()

    return k


def kernel(x, seq_lengths):
    x2 = x.reshape(B * S, D)
    lens = jnp.repeat(seq_lengths.astype(jnp.int32), LANES)
    out = _make_sc_reverse()(x2, lens)
    return out.reshape(B, S, D)


# speedup vs baseline: 25.2527x; 1.0429x over previous
import functools

import jax
import jax.numpy as jnp
from jax import lax
from jax.experimental import pallas as pl
from jax.experimental.pallas import tpu as pltpu
from jax.experimental.pallas import tpu_sc as plsc

B, S, D = 16, 4096, 1024
NC, NS, LANES = 2, 16, 16
NW = NC * NS
ROWS_PER_W = (B * S) // NW
CH = 16
NCHUNK = ROWS_PER_W // CH
NBUF = 3


def _make_sc_reverse():
    mesh = plsc.VectorSubcoreMesh(core_axis_name="c", subcore_axis_name="s")

    @functools.partial(
        pl.kernel,
        mesh=mesh,
        out_type=jax.ShapeDtypeStruct((B * S, D), jnp.float32),
        scratch_types=[
            pltpu.VMEM((B * LANES,), jnp.int32),
            pltpu.VMEM((ROWS_PER_W,), jnp.int32),
            pltpu.VMEM((NBUF, CH, D), jnp.float32),
            pltpu.VMEM_SHARED((NS, NBUF, CH, D), jnp.float32),
            pltpu.SemaphoreType.DMA((NBUF,)),
            pltpu.SemaphoreType.DMA((NBUF,)),
            pltpu.SemaphoreType.DMA((NBUF,)),
            pltpu.SemaphoreType.DMA((NBUF,)),
        ],
    )
    def k(x_hbm, len_hbm, out_hbm, len_v, idx_v, vbuf, sbuf_all,
          sgv, ssv, sgs, sss):
        sid = lax.axis_index("s")
        sbuf = sbuf_all.at[sid]
        wid = sid * NC + lax.axis_index("c")
        b = wid // 2
        s_base = (wid % 2) * ROWS_PER_W
        row0 = b * S

        pltpu.sync_copy(len_hbm, len_v)
        lvec = len_v[pl.ds(b * LANES, LANES)][0]
        out0 = row0 + s_base

        def rev(c):
            return s_base + c * CH < lvec

        def build(cc):
            for r in range(CH // LANES):
                p = s_base + cc * CH + r * LANES + lax.iota(jnp.int32, LANES)
                src = jnp.where(p < lvec, lvec - 1 - p, p) + row0
                idx_v[pl.ds(cc * CH + r * LANES, LANES)] = src

        def g_rev(c):
            j = c % NBUF
            return pltpu.make_async_copy(
                x_hbm.at[idx_v.at[pl.ds(c * CH, CH)]], vbuf.at[j], sgv.at[j])

        def s_rev(c):
            j = c % NBUF
            return pltpu.make_async_copy(
                vbuf.at[j], out_hbm.at[pl.ds(out0 + c * CH, CH)], ssv.at[j])

        def g_id(c):
            j = c % NBUF
            return pltpu.make_async_copy(
                x_hbm.at[pl.ds(out0 + c * CH, CH)], sbuf.at[j], sgs.at[j])

        def s_id(c):
            j = c % NBUF
            return pltpu.make_async_copy(
                sbuf.at[j], out_hbm.at[pl.ds(out0 + c * CH, CH)], sss.at[j])

        def start_gather(c):
            @pl.when(rev(c))
            def _():
                build(c)
                g_rev(c).start()

            @pl.when(jnp.logical_not(rev(c)))
            def _():
                g_id(c).start()

        def wait_gather(c):
            @pl.when(rev(c))
            def _():
                g_rev(c).wait()

            @pl.when(jnp.logical_not(rev(c)))
            def _():
                g_id(c).wait()

        def start_scatter(c):
            @pl.when(rev(c))
            def _():
                s_rev(c).start()

            @pl.when(jnp.logical_not(rev(c)))
            def _():
                s_id(c).start()

        def wait_scatter(c):
            @pl.when(rev(c))
            def _():
                s_rev(c).wait()

            @pl.when(jnp.logical_not(rev(c)))
            def _():
                s_id(c).wait()

        start_gather(0)
        start_gather(1)

        def step(c, _):
            wait_gather(c)
            start_scatter(c)

            @pl.when(c >= 1)
            def _():
                wait_scatter(c - 1)

            @pl.when(c + 2 < NCHUNK)
            def _():
                start_gather(c + 2)

            return 0

        lax.fori_loop(0, NCHUNK, step, 0)
        wait_scatter(NCHUNK - 1)

    return k


def kernel(x, seq_lengths):
    x2 = x.reshape(B * S, D)
    lens = jnp.repeat(seq_lengths.astype(jnp.int32), LANES)
    out = _make_sc_reverse()(x2, lens)
    return out.reshape(B, S, D)


# speedup vs baseline: 25.2731x; 1.0008x over previous
import functools

import jax
import jax.numpy as jnp
from jax import lax
from jax.experimental import pallas as pl
from jax.experimental.pallas import tpu as pltpu
from jax.experimental.pallas import tpu_sc as plsc

B, S, D = 16, 4096, 1024
NC, NS, LANES = 2, 16, 16
NW = NC * NS
ROWS_PER_W = (B * S) // NW
CH = 16
NCHUNK = ROWS_PER_W // CH
NBUF = 3


def _make_sc_reverse():
    mesh = plsc.VectorSubcoreMesh(core_axis_name="c", subcore_axis_name="s")

    @functools.partial(
        pl.kernel,
        mesh=mesh,
        out_type=jax.ShapeDtypeStruct((B * S, D), jnp.float32),
        scratch_types=[
            pltpu.VMEM((B * LANES,), jnp.int32),
            pltpu.VMEM((ROWS_PER_W,), jnp.int32),
            pltpu.VMEM((NBUF, CH, D), jnp.float32),
            pltpu.VMEM_SHARED((NS, NBUF, CH, D), jnp.float32),
            pltpu.SemaphoreType.DMA((NBUF,)),
            pltpu.SemaphoreType.DMA((NBUF,)),
            pltpu.SemaphoreType.DMA((NBUF,)),
            pltpu.SemaphoreType.DMA((NBUF,)),
        ],
    )
    def k(x_hbm, len_hbm, out_hbm, len_v, idx_v, vbuf, sbuf_all,
          sgv, ssv, sgs, sss):
        sid = lax.axis_index("s")
        sbuf = sbuf_all.at[sid]
        wid = sid * NC + lax.axis_index("c")
        b = wid // 2
        s_base = (wid % 2) * ROWS_PER_W
        row0 = b * S

        pltpu.sync_copy(len_hbm, len_v)
        lvec = len_v[pl.ds(b * LANES, LANES)][0]
        out0 = row0 + s_base

        d_rows = jnp.maximum(lvec - s_base, 0)
        n_rev = jnp.minimum((d_rows + CH - 1) // CH, NCHUNK)
        n_id = NCHUNK - n_rev

        def build(cc):
            for r in range(CH // LANES):
                p = s_base + cc * CH + r * LANES + lax.iota(jnp.int32, LANES)
                src = jnp.where(p < lvec, lvec - 1 - p, p) + row0
                idx_v[pl.ds(cc * CH + r * LANES, LANES)] = src

        def g_rev(c):
            j = c % NBUF
            return pltpu.make_async_copy(
                x_hbm.at[idx_v.at[pl.ds(c * CH, CH)]], vbuf.at[j], sgv.at[j])

        def s_rev(c):
            j = c % NBUF
            return pltpu.make_async_copy(
                vbuf.at[j], out_hbm.at[pl.ds(out0 + c * CH, CH)], ssv.at[j])

        def g_id(t, c):
            j = t % NBUF
            return pltpu.make_async_copy(
                x_hbm.at[pl.ds(out0 + c * CH, CH)], sbuf.at[j], sgs.at[j])

        def s_id(t, c):
            j = t % NBUF
            return pltpu.make_async_copy(
                sbuf.at[j], out_hbm.at[pl.ds(out0 + c * CH, CH)], sss.at[j])

        def a_start(t):
            @pl.when(t < n_rev)
            def _():
                build(t)
                g_rev(t).start()

        def b_start(t):
            @pl.when(t < n_id)
            def _():
                g_id(t, NCHUNK - 1 - t).start()

        a_start(0)
        b_start(0)
        a_start(1)
        b_start(1)

        def step(t, _):
            @pl.when(t < n_rev)
            def _():
                g_rev(t).wait()
                s_rev(t).start()

            @pl.when(t < n_id)
            def _():
                g_id(t, NCHUNK - 1 - t).wait()
                s_id(t, NCHUNK - 1 - t).start()

            @pl.when(jnp.logical_and(t >= 1, t <= n_rev))
            def _():
                s_rev(t - 1).wait()

            @pl.when(jnp.logical_and(t >= 1, t <= n_id))
            def _():
                s_id(t - 1, NCHUNK - t).wait()

            a_start(t + 2)
            b_start(t + 2)
            return 0

        lax.fori_loop(0, NCHUNK, step, 0)

        @pl.when(n_rev >= NCHUNK)
        def _():
            s_rev(NCHUNK - 1).wait()

        @pl.when(n_id >= NCHUNK)
        def _():
            s_id(NCHUNK - 1, 0).wait()

    return k


def kernel(x, seq_lengths):
    x2 = x.reshape(B * S, D)
    lens = jnp.repeat(seq_lengths.astype(jnp.int32), LANES)
    out = _make_sc_reverse()(x2, lens)
    return out.reshape(B, S, D)
